# baseline (speedup 1.0000x reference)
import jax
import jax.numpy as jnp
import numpy as np
from jax import lax
from jax.experimental import pallas as pl
from jax.experimental.pallas import tpu as pltpu
from jax.experimental.pallas import tpu_sc as plsc

_MAXLEN = 8192
_DIM = 64
_BATCH = 64

_NC = 2
_NS = 16
_NW = _NC * _NS

_SLAB = _MAXLEN // _NW
_NCHUNK = _BATCH
_NBUF = 3
_LANES = 16
_VPR = _DIM // _LANES


def _sc_body(x_hbm, pos_hbm, out_hbm,
             bufs, pos_buf, lsem0, lsem1, lsem2, ssem0, ssem1, ssem2):
    lsems = (lsem0, lsem1, lsem2)
    ssems = (ssem0, ssem1, ssem2)

    wid = lax.axis_index("s") * _NC + lax.axis_index("c")
    base_pos = wid * _SLAB

    pltpu.sync_copy(pos_hbm.at[wid], pos_buf)

    def row0(c):
        return c * _MAXLEN + base_pos

    def load(c, k):
        pltpu.async_copy(x_hbm.at[pl.ds(row0(c), _SLAB)], bufs.at[k],
                         lsems[k])

    def wait_load(c, k):
        pltpu.make_async_copy(x_hbm.at[pl.ds(row0(c), _SLAB)], bufs.at[k],
                              lsems[k]).wait()

    def store(c, k):
        pltpu.async_copy(bufs.at[k], out_hbm.at[pl.ds(row0(c), _SLAB)],
                         ssems[k])

    def wait_store(c, k):
        pltpu.make_async_copy(bufs.at[k], out_hbm.at[pl.ds(row0(c), _SLAB)],
                              ssems[k]).wait()

    load(0, 0)
    for c in range(_NCHUNK):
        k = c % _NBUF
        wait_load(c, k)
        if c + 1 < _NCHUNK:
            if c >= 2:
                wait_store(c - 2, (k + 1) % _NBUF)
            load(c + 1, (k + 1) % _NBUF)

        @plsc.parallel_loop(0, _SLAB, unroll=4)
        def _(r):
            rr = lax.rem(r, _SLAB // 2)
            half = (r // (_SLAB // 2)) * _DIM
            for li in range(_VPR):
                sl = pl.ds(li * _LANES, _LANES)
                plsc.addupdate(bufs.at[k, r, sl],
                               pos_buf[rr, pl.ds(half + li * _LANES,
                                                 _LANES)])

        store(c, k)
    wait_store(_NCHUNK - 2, (_NCHUNK - 2) % _NBUF)
    wait_store(_NCHUNK - 1, (_NCHUNK - 1) % _NBUF)


_sc_call = pl.kernel(
    _sc_body,
    out_type=jax.ShapeDtypeStruct((_BATCH * _MAXLEN, _DIM), jnp.float32),
    mesh=plsc.VectorSubcoreMesh(core_axis_name="c", subcore_axis_name="s"),
    scratch_types=[
        pltpu.VMEM((_NBUF, _SLAB, _DIM), jnp.float32),
        pltpu.VMEM((_SLAB // 2, 2 * _DIM), jnp.float32),
        pltpu.SemaphoreType.DMA,
        pltpu.SemaphoreType.DMA,
        pltpu.SemaphoreType.DMA,
        pltpu.SemaphoreType.DMA,
        pltpu.SemaphoreType.DMA,
        pltpu.SemaphoreType.DMA,
    ],
)


@jax.jit
def kernel(x, pos_table):
    slabs = pos_table.reshape(_NW, 2, _SLAB // 2, _DIM)
    pos_packed = jnp.concatenate([slabs[:, 0], slabs[:, 1]], axis=-1)
    out = _sc_call(x.reshape(_BATCH * _MAXLEN, _DIM), pos_packed)
    return out.reshape(x.shape)


# speedup vs baseline: 1.0326x; 1.0326x over previous
import jax
import jax.numpy as jnp
import numpy as np
from jax import lax
from jax.experimental import pallas as pl
from jax.experimental.pallas import tpu as pltpu
from jax.experimental.pallas import tpu_sc as plsc

_MAXLEN = 8192
_DIM = 64
_BATCH = 64

_NC = 2
_NS = 16
_NW = _NC * _NS

_SLAB = _MAXLEN // _NW
_CP = 128
_CPB = _SLAB // _CP
_NCHUNK = _BATCH * _CPB
_NBUF = 6
_LEAD = 3
_LANES = 16
_VPR = _DIM // _LANES


def _sc_body(x_hbm, pos_hbm, out_hbm,
             bufs, pos_buf,
             lsem0, lsem1, lsem2, lsem3, lsem4, lsem5,
             ssem0, ssem1, ssem2, ssem3, ssem4, ssem5):
    lsems = (lsem0, lsem1, lsem2, lsem3, lsem4, lsem5)
    ssems = (ssem0, ssem1, ssem2, ssem3, ssem4, ssem5)

    wid = lax.axis_index("s") * _NC + lax.axis_index("c")
    base_pos = wid * _SLAB

    pltpu.sync_copy(pos_hbm.at[wid], pos_buf)

    def row0(c):
        return (c // _CPB) * _MAXLEN + base_pos + (c % _CPB) * _CP

    def load(c, k):
        pltpu.async_copy(x_hbm.at[pl.ds(row0(c), _CP)], bufs.at[k],
                         lsems[k])

    def wait_load(c, k):
        pltpu.make_async_copy(x_hbm.at[pl.ds(row0(c), _CP)], bufs.at[k],
                              lsems[k]).wait()

    def store(c, k):
        pltpu.async_copy(bufs.at[k], out_hbm.at[pl.ds(row0(c), _CP)],
                         ssems[k])

    def wait_store(c, k):
        pltpu.make_async_copy(bufs.at[k], out_hbm.at[pl.ds(row0(c), _CP)],
                              ssems[k]).wait()

    def chunk_body(c, k):
        j = k % _CPB
        wait_load(c, k)
        @pl.when(c + _LEAD < _NCHUNK)
        def _():
            @pl.when(c >= _LEAD)
            def _():
                wait_store(c - _LEAD, (k + _LEAD) % _NBUF)

            load(c + _LEAD, (k + _LEAD) % _NBUF)

        @plsc.parallel_loop(0, _CP, unroll=4)
        def _(r):
            for li in range(_VPR):
                plsc.addupdate(bufs.at[k, r, pl.ds(li * _LANES, _LANES)],
                               pos_buf[r, pl.ds(j * _DIM + li * _LANES,
                                                _LANES)])

        store(c, k)

    for c in range(_LEAD):
        load(c, c)

    _NFULL = (_NCHUNK // _NBUF) * _NBUF

    def step(t, carry):
        for k in range(_NBUF):
            chunk_body(t * _NBUF + k, k)
        return carry

    lax.fori_loop(0, _NFULL // _NBUF, step, 0)
    for c in range(_NFULL, _NCHUNK):
        chunk_body(c, c % _NBUF)
    for c in range(_NCHUNK - 2 * _LEAD, _NCHUNK):
        wait_store(c, c % _NBUF)


_sc_call = pl.kernel(
    _sc_body,
    out_type=jax.ShapeDtypeStruct((_BATCH * _MAXLEN, _DIM), jnp.float32),
    mesh=plsc.VectorSubcoreMesh(core_axis_name="c", subcore_axis_name="s"),
    scratch_types=[
        pltpu.VMEM((_NBUF, _CP, _DIM), jnp.float32),
        pltpu.VMEM((_SLAB // 2, 2 * _DIM), jnp.float32),
        pltpu.SemaphoreType.DMA,
        pltpu.SemaphoreType.DMA,
        pltpu.SemaphoreType.DMA,
        pltpu.SemaphoreType.DMA,
        pltpu.SemaphoreType.DMA,
        pltpu.SemaphoreType.DMA,
        pltpu.SemaphoreType.DMA,
        pltpu.SemaphoreType.DMA,
        pltpu.SemaphoreType.DMA,
        pltpu.SemaphoreType.DMA,
        pltpu.SemaphoreType.DMA,
        pltpu.SemaphoreType.DMA,
    ],
)


@jax.jit
def kernel(x, pos_table):
    slabs = pos_table.reshape(_NW, 2, _SLAB // 2, _DIM)
    pos_packed = jnp.concatenate([slabs[:, 0], slabs[:, 1]], axis=-1)
    out = _sc_call(x.reshape(_BATCH * _MAXLEN, _DIM), pos_packed)
    return out.reshape(x.shape)
